# Initial kernel scaffold; baseline (speedup 1.0000x reference)
import jax
import jax.numpy as jnp
from jax.experimental import pallas as pl


def kernel(naics_2_digit, naics_3_digit, naics_4_digit, naics_5_digit, naics_6_digit, E2, D3, D4, D5, D6, W, b):
    raise NotImplementedError("write your pallas kernel here")


# speedup vs baseline: 3.1153x; 3.1153x over previous
import functools

import jax
import jax.numpy as jnp
from jax.experimental import pallas as pl
from jax.experimental.pallas import tpu as pltpu

_B = 16384
_D = 64
_BB = 512
_NB = _B // _BB
_NU = (25, 100, 400, 750, 1200)


def _onehot(idx_col, n):
    iota = jax.lax.broadcasted_iota(jnp.int32, (_BB, n), 1)
    return (idx_col == iota).astype(jnp.float32)


def _normalize(e):
    s = jnp.sum(e * e, axis=1, keepdims=True)
    return e * jax.lax.rsqrt(jnp.maximum(s, 1e-24))


def _chain_body(i2, i3, i4, i5, i6, e2t, d3t, d4t, d5t, d6t, w, b, out):
    dot = functools.partial(jnp.dot, preferred_element_type=jnp.float32)
    e = _normalize(dot(_onehot(i2[0], _NU[0]), e2t[...]))
    e = _normalize(e + dot(_onehot(i3[0], _NU[1]), d3t[...]))
    e = _normalize(e + dot(_onehot(i4[0], _NU[2]), d4t[...]))
    e = _normalize(e + dot(_onehot(i5[0], _NU[3]), d5t[...]))
    e = _normalize(e + dot(_onehot(i6[0], _NU[4]), d6t[...]))
    logit = jnp.sum(e * w[...], axis=1, keepdims=True) + b[...]
    out[...] = 1.0 / (1.0 + jnp.exp(-logit))


def kernel(naics_2_digit, naics_3_digit, naics_4_digit, naics_5_digit,
           naics_6_digit, E2, D3, D4, D5, D6, W, b):
    idxs = [x.astype(jnp.int32).reshape(_NB, _BB, 1)
            for x in (naics_2_digit, naics_3_digit, naics_4_digit,
                      naics_5_digit, naics_6_digit)]
    wrow = W.reshape(1, _D)
    b2 = b.reshape(1, 1)

    idx_spec = pl.BlockSpec((1, _BB, 1), lambda i: (i, 0, 0))
    full = lambda shape: pl.BlockSpec(shape, lambda i: (0,) * len(shape))

    out = pl.pallas_call(
        _chain_body,
        grid=(_NB,),
        in_specs=[idx_spec] * 5 + [
            full(E2.shape), full(D3.shape), full(D4.shape),
            full(D5.shape), full(D6.shape), full((1, _D)), full((1, 1)),
        ],
        out_specs=pl.BlockSpec((_BB, 1), lambda i: (i, 0)),
        out_shape=jax.ShapeDtypeStruct((_B, 1), jnp.float32),
    )(*idxs, E2, D3, D4, D5, D6, wrow, b2)
    return out


# speedup vs baseline: 4.4608x; 1.4319x over previous
import dataclasses
import functools

import jax
import jax.numpy as jnp
from jax import lax
from jax.experimental import pallas as pl
from jax.experimental.pallas import tpu as pltpu
from jax.experimental.pallas import tpu_sc as plsc

_B = 16384
_D = 64
_NU = (25, 100, 400, 750, 1200)
_PAIRS = tuple((a, c) for a in range(5) for c in range(a + 1, 5))
_NW = 32
_CH = _B // _NW
_NGRP = _CH // 16
_AUXW = 1280
_EPS2 = 1e-24


def _gram_body(*refs):
    a_refs = refs[0:4]
    t_refs = refs[4:9]
    wcol = refs[9]
    g_refs = refs[10:20]
    aux_ref = refs[20]

    aux_ref[...] = jnp.zeros((10, _AUXW), jnp.float32)
    for p, (a, c) in enumerate(_PAIRS):
        g_refs[p][...] = jnp.dot(a_refs[a][...], t_refs[c][...],
                                 preferred_element_type=jnp.float32)
    w = wcol[...]
    for k in range(5):
        t = t_refs[k][...]
        aux_ref[k, pl.ds(0, _NU[k])] = jnp.sum(t * t, axis=0)
        aux_ref[5 + k, pl.ds(0, _NU[k])] = jnp.sum(t * w, axis=0)


def _gram_tc(tables, W):
    full = lambda shape: pl.BlockSpec(shape, lambda: (0,) * len(shape))
    a_in = tables[:4]
    t_in = [t.T for t in tables]
    out_shapes = tuple(
        jax.ShapeDtypeStruct((_NU[a], _NU[c]), jnp.float32) for a, c in _PAIRS
    ) + (jax.ShapeDtypeStruct((10, _AUXW), jnp.float32),)
    ins = list(a_in) + t_in + [W]
    return pl.pallas_call(
        _gram_body,
        in_specs=[full(x.shape) for x in ins],
        out_specs=tuple(full(s.shape) for s in out_shapes),
        out_shape=out_shapes,
    )(*ins)


def _rsqrt(s):
    i = plsc.bitcast(s, jnp.int32)
    y = plsc.bitcast(jnp.int32(0x5F3759DF) - (i >> 1), jnp.float32)
    xh = s * 0.5
    for _ in range(3):
        y = y * (1.5 - xh * y * y)
    return y


def _sc_body(*refs):
    idx_h = refs[0:5]
    g_h = refs[5:15]
    aux_h, b_h = refs[15], refs[16]
    out_h = refs[17]
    idx_v = refs[18:23]
    aux_v, b_v = refs[23], refs[24]
    comb_v = refs[25:35]
    gath_v = refs[35:45]
    out_v = refs[45]
    sem = refs[46]

    wid = lax.axis_index("s") * 2 + lax.axis_index("c")
    base = wid * _CH

    for k in range(5):
        pltpu.sync_copy(idx_h[k].at[pl.ds(base, _CH)], idx_v[k])
    pltpu.sync_copy(aux_h, aux_v)
    pltpu.sync_copy(b_h, b_v)

    @pl.loop(0, _NGRP)
    def _(t):
        sl = pl.ds(t * 16, 16)
        iv = [idx_v[k][sl] for k in range(5)]
        for p, (a, c) in enumerate(_PAIRS):
            comb_v[p][sl] = iv[a] * _NU[c] + iv[c]

    copies = []
    for p in range(10):
        for j in range(4):
            csl = pl.ds(j * 128, 128)
            copies.append(pltpu.async_copy(
                g_h[p].at[comb_v[p].at[csl]], gath_v[p].at[csl], sem))
    for c in copies:
        c.wait()

    @pl.loop(0, _NGRP)
    def _(t):
        sl = pl.ds(t * 16, 16)
        iv = [idx_v[k][sl] for k in range(5)]
        nsq = [plsc.load_gather(aux_v, [jnp.full((16,), k, jnp.int32), iv[k]])
               for k in range(5)]
        tw = [plsc.load_gather(aux_v, [jnp.full((16,), 5 + k, jnp.int32), iv[k]])
              for k in range(5)]
        g23, g24, g25, g26, g34, g35, g36, g45, g46, g56 = (
            gath_v[p][sl] for p in range(10))

        inv2 = _rsqrt(jnp.maximum(nsq[0], _EPS2))
        h2 = nsq[0] * inv2 * inv2
        s3 = h2 + 2.0 * (g23 * inv2) + nsq[1]
        inv3 = _rsqrt(jnp.maximum(s3, _EPS2))
        h3 = s3 * inv3 * inv3
        x5 = g25 * inv2 + g35
        x6 = g26 * inv2 + g36
        a4 = (g24 * inv2 + g34) * inv3
        s4 = h3 + 2.0 * a4 + nsq[2]
        inv4 = _rsqrt(jnp.maximum(s4, _EPS2))
        h4 = s4 * inv4 * inv4
        a5 = (x5 * inv3 + g45) * inv4
        s5 = h4 + 2.0 * a5 + nsq[3]
        inv5 = _rsqrt(jnp.maximum(s5, _EPS2))
        h5 = s5 * inv5 * inv5
        a6 = ((x6 * inv3 + g46) * inv4 + g56) * inv5
        s6 = h5 + 2.0 * a6 + nsq[4]
        inv6 = _rsqrt(jnp.maximum(s6, _EPS2))
        logit = ((((tw[0] * inv2 + tw[1]) * inv3 + tw[2]) * inv4 + tw[3])
                 * inv5 + tw[4]) * inv6 + b_v[...]
        out_v[sl] = 1.0 / (1.0 + jnp.exp(-logit))

    pltpu.sync_copy(out_v, out_h.at[pl.ds(base, _CH)])


def _sc_call(idxs, g_flats, aux, bvec):
    mesh = plsc.VectorSubcoreMesh(core_axis_name="c", subcore_axis_name="s")
    scratch = (
        [pltpu.VMEM((_CH,), jnp.int32) for _ in range(5)]
        + [pltpu.VMEM((10, _AUXW), jnp.float32), pltpu.VMEM((16,), jnp.float32)]
        + [pltpu.VMEM((_CH,), jnp.int32) for _ in range(10)]
        + [pltpu.VMEM((_CH,), jnp.float32) for _ in range(10)]
        + [pltpu.VMEM((_CH,), jnp.float32), pltpu.SemaphoreType.DMA]
    )
    cp = pltpu.CompilerParams()
    if "needs_layout_passes" in pltpu.CompilerParams.__dataclass_fields__:
        cp = dataclasses.replace(cp, needs_layout_passes=False)
    run = pl.kernel(
        _sc_body,
        out_type=jax.ShapeDtypeStruct((_B,), jnp.float32),
        mesh=mesh,
        scratch_types=scratch,
        compiler_params=cp,
    )
    return run(*idxs, *g_flats, aux, bvec)


def kernel(naics_2_digit, naics_3_digit, naics_4_digit, naics_5_digit,
           naics_6_digit, E2, D3, D4, D5, D6, W, b):
    idxs = [x.astype(jnp.int32)
            for x in (naics_2_digit, naics_3_digit, naics_4_digit,
                      naics_5_digit, naics_6_digit)]
    tables = [E2, D3, D4, D5, D6]
    outs = _gram_tc(tables, W.astype(jnp.float32))
    g_flats = [g.reshape(-1) for g in outs[:10]]
    aux = outs[10]
    bvec = jnp.broadcast_to(b.astype(jnp.float32), (16,))
    out = _sc_call(idxs, g_flats, aux, bvec)
    return out.reshape(_B, 1)
